# Initial kernel scaffold; baseline (speedup 1.0000x reference)
import jax
import jax.numpy as jnp
from jax.experimental import pallas as pl


def kernel(x, edge_index, edge_type, W1, root1, bias1, W2, root2, bias2):
    raise NotImplementedError("write your pallas kernel here")


# speedup vs baseline: 3.8301x; 3.8301x over previous
import functools

import jax
import jax.numpy as jnp
from jax import lax
from jax.experimental import pallas as pl
from jax.experimental.pallas import tpu as pltpu
from jax.experimental.pallas import tpu_sc as plsc

N = 10000
R = 8
DIN = 16
H = 512
E = 160000
NR = N * R
NRP = 81920
NC = 2
NS = 16
L = 16
HC = 4
CW = H // HC
HXROWS = (R + 1) * N

_MESH = dict(core_axis_name="c", subcore_axis_name="s",
             num_cores=NC, num_subcores=NS)

_CNT_R = NRP // (NC * NS)
_CA_EB = 1600
_CA_NB = E // _CA_EB


def _sc_counts_body(dst_ref, rel_ref, winv_ref, cnt, db, rb):
    core = lax.axis_index("c")
    sub = lax.axis_index("s")
    wid = sub * NC + core
    base = wid * _CNT_R
    zeros = jnp.zeros((L,), jnp.float32)
    ones = jnp.ones((L,), jnp.float32)

    def zero_body(i, c):
        cnt[pl.ds(i * L, L)] = zeros
        return c
    lax.fori_loop(0, _CNT_R // L, zero_body, 0)

    def blk(b, c):
        e0 = b * _CA_EB
        pltpu.sync_copy(dst_ref.at[pl.ds(e0, _CA_EB)], db)
        pltpu.sync_copy(rel_ref.at[pl.ds(e0, _CA_EB)], rb)
        for i in range(_CA_EB // L):
            sl = pl.ds(i * L, L)
            loc = db[sl] * R + rb[sl] - base
            msk = (loc >= 0) & (loc < _CNT_R)
            plsc.addupdate_scatter(cnt, [jnp.where(msk, loc, 0)], ones,
                                   mask=msk)
        return c
    lax.fori_loop(0, _CA_NB, blk, 0)

    def winv_body(i, c):
        sl = pl.ds(i * L, L)
        cnt[sl] = 1.0 / jnp.maximum(cnt[sl], 1.0)
        return c
    lax.fori_loop(0, _CNT_R // L, winv_body, 0)
    pltpu.sync_copy(cnt, winv_ref.at[pl.ds(base, _CNT_R)])


_sc_counts = functools.partial(
    pl.kernel,
    out_type=jax.ShapeDtypeStruct((NRP,), jnp.float32),
    mesh=plsc.VectorSubcoreMesh(**_MESH),
    compiler_params=pltpu.CompilerParams(needs_layout_passes=False,
                                         use_tc_tiling_on_sc=False),
    scratch_types=[
        pltpu.VMEM((_CNT_R,), jnp.float32),
        pltpu.VMEM((_CA_EB,), jnp.int32),
        pltpu.VMEM((_CA_EB,), jnp.int32),
    ],
)(_sc_counts_body)

_EPT = E // NS
_B_EB = 2000
_B_NB = _EPT // _B_EB
_SB = 80
_B_NS = _B_EB // _SB
_VPS = _SB // L
_SUMS_R = NR // NS


def _sc_sums_body(src_ref, dst_ref, rel_ref, x_ref, winv_ref,
                  sums_ref, wedge_ref,
                  acc, sb, db, rb, sb2, ib2, rows, w80, wo, zbuf, sem):
    core = lax.axis_index("c")
    sub = lax.axis_index("s")
    zeros = jnp.zeros((L,), jnp.float32)

    @pl.when(core == 0)
    def _():
        base = sub * _SUMS_R

        def zb(i, c):
            zbuf[i] = zeros
            return c
        lax.fori_loop(0, 500, zb, 0)

        def zdma(j, c):
            pltpu.sync_copy(zbuf, acc.at[pl.ds(base + j * 500, 500)])
            return c
        lax.fori_loop(0, _SUMS_R // 500, zdma, 0)
        plsc.subcore_barrier()

        def blk(b, c):
            e0 = sub * _EPT + b * _B_EB
            pltpu.sync_copy(src_ref.at[pl.ds(e0, _B_EB)], sb)
            pltpu.sync_copy(dst_ref.at[pl.ds(e0, _B_EB)], db)
            pltpu.sync_copy(rel_ref.at[pl.ds(e0, _B_EB)], rb)
            for i in range(_B_EB // L):
                sl = pl.ds(i * L, L)
                dsl = pl.ds((i % _VPS) * L, L)
                ib2[i // _VPS, dsl] = db[sl] * R + rb[sl]
                sb2[i // _VPS, dsl] = sb[sl]
            for j in range(_B_NS):
                pltpu.async_copy(x_ref.at[sb2.at[j]], rows, sem).wait()
                pltpu.sync_copy(rows, acc.at[ib2.at[j]], add=True)
            return c
        lax.fori_loop(0, _B_NB, blk, 0)
        plsc.subcore_barrier()
        pltpu.sync_copy(acc.at[pl.ds(base, _SUMS_R)],
                        sums_ref.at[pl.ds(base, _SUMS_R)])

    @pl.when(core == 1)
    def _():
        def blk(b, c):
            e0 = sub * _EPT + b * _B_EB
            pltpu.sync_copy(dst_ref.at[pl.ds(e0, _B_EB)], db)
            pltpu.sync_copy(rel_ref.at[pl.ds(e0, _B_EB)], rb)
            for i in range(_B_EB // L):
                sl = pl.ds(i * L, L)
                dsl = pl.ds((i % _VPS) * L, L)
                ib2[i // _VPS, dsl] = db[sl] * R + rb[sl]
            for j in range(_B_NS):
                pltpu.async_copy(winv_ref.at[ib2.at[j]], w80, sem).wait()
                for t in range(_VPS):
                    wo[pl.ds(j * _SB + t * L, L)] = w80[pl.ds(t * L, L)]
            pltpu.sync_copy(wo, wedge_ref.at[pl.ds(e0, _B_EB)])
            return c
        lax.fori_loop(0, _B_NB, blk, 0)


_sc_sums = functools.partial(
    pl.kernel,
    out_type=(jax.ShapeDtypeStruct((NR, DIN), jnp.float32),
              jax.ShapeDtypeStruct((E,), jnp.float32)),
    mesh=plsc.VectorSubcoreMesh(**_MESH),
    compiler_params=pltpu.CompilerParams(needs_layout_passes=False,
                                         use_tc_tiling_on_sc=False),
    scratch_types=[
        pltpu.VMEM_SHARED((NR, DIN), jnp.float32),
        pltpu.VMEM((_B_EB,), jnp.int32),
        pltpu.VMEM((_B_EB,), jnp.int32),
        pltpu.VMEM((_B_EB,), jnp.int32),
        pltpu.VMEM((_B_NS, _SB), jnp.int32),
        pltpu.VMEM((_B_NS, _SB), jnp.int32),
        pltpu.VMEM((_SB, DIN), jnp.float32),
        pltpu.VMEM((_SB,), jnp.float32),
        pltpu.VMEM((_B_EB,), jnp.float32),
        pltpu.VMEM((500, DIN), jnp.float32),
        pltpu.SemaphoreType.DMA,
    ],
)(_sc_sums_body)

_BN = 1000


def _tc1_body(sums_ref, winv_ref, x_ref, w1_ref, r1_ref, b1_ref, out_ref):
    means = sums_ref[...] * winv_ref[...]
    h = jnp.dot(means, w1_ref[...], preferred_element_type=jnp.float32)
    h = h + jnp.dot(x_ref[...], r1_ref[...], preferred_element_type=jnp.float32)
    h = h + b1_ref[...]
    out_ref[...] = jnp.maximum(h, 0.0)


def _tc_layer1(sums2d, winv16, x, w1f, root1, bias1):
    return pl.pallas_call(
        _tc1_body,
        grid=(N // _BN,),
        in_specs=[
            pl.BlockSpec((_BN, R * DIN), lambda i: (i, 0)),
            pl.BlockSpec((_BN, R * DIN), lambda i: (i, 0)),
            pl.BlockSpec((_BN, DIN), lambda i: (i, 0)),
            pl.BlockSpec((R * DIN, H), lambda i: (0, 0)),
            pl.BlockSpec((DIN, H), lambda i: (0, 0)),
            pl.BlockSpec((1, H), lambda i: (0, 0)),
        ],
        out_specs=pl.BlockSpec((_BN, H), lambda i: (i, 0)),
        out_shape=jax.ShapeDtypeStruct((N, H), jnp.float32),
    )(sums2d, winv16, x, w1f, root1, bias1)


def _tc2_body(h1_ref, w_ref, b2_ref, out_ref):
    r = pl.program_id(2)
    v = jnp.dot(h1_ref[...], w_ref[...].reshape(H, CW),
                preferred_element_type=jnp.float32)
    v = v + jnp.where(r == R, b2_ref[0], 0.0)
    out_ref[...] = v[None]


def _tc_hall(h1, w2x, bias2c):
    return pl.pallas_call(
        _tc2_body,
        grid=(N // _BN, HC, R + 1),
        in_specs=[
            pl.BlockSpec((_BN, H), lambda nb, c, r: (nb, 0)),
            pl.BlockSpec((1, 1, H, CW), lambda nb, c, r: (c, r, 0, 0)),
            pl.BlockSpec((1, 1, CW), lambda nb, c, r: (c, 0, 0)),
        ],
        out_specs=pl.BlockSpec((1, _BN, CW),
                               lambda nb, c, r: (c, r * (N // _BN) + nb, 0)),
        out_shape=jax.ShapeDtypeStruct((HC, HXROWS, CW), jnp.float32),
    )(h1, w2x, bias2c)

_ROWS_PT = N // NS


def _sc_layer2_body(src_ref, dst_ref, rel_ref, we_ref, hx_ref, agg_ref,
                    acc2, sb, db, rb, wb, gb2, db2, rows_a, rows_b,
                    sem_a, sem_b):
    core = lax.axis_index("c")
    sub = lax.axis_index("s")

    for chunk in range(HC):
        def _chunk(chunk=chunk):
            pltpu.sync_copy(
                hx_ref.at[pl.ds(chunk * HXROWS + R * N + sub * _ROWS_PT,
                                _ROWS_PT)],
                acc2.at[pl.ds(sub * _ROWS_PT, _ROWS_PT)])
            plsc.subcore_barrier()

            def blk(b, c):
                e0 = sub * _EPT + b * _B_EB
                pltpu.sync_copy(src_ref.at[pl.ds(e0, _B_EB)], sb)
                pltpu.sync_copy(dst_ref.at[pl.ds(e0, _B_EB)], db)
                pltpu.sync_copy(rel_ref.at[pl.ds(e0, _B_EB)], rb)
                pltpu.sync_copy(we_ref.at[pl.ds(e0, _B_EB)], wb)
                for i in range(_B_EB // L):
                    sl = pl.ds(i * L, L)
                    dsl = pl.ds((i % _VPS) * L, L)
                    gb2[i // _VPS, dsl] = rb[sl] * N + sb[sl] + chunk * HXROWS
                    db2[i // _VPS, dsl] = db[sl]
                bufs = (rows_a, rows_b)
                sems = (sem_a, sem_b)
                cp = pltpu.async_copy(hx_ref.at[gb2.at[0]], rows_a, sem_a)
                for j in range(_B_NS):
                    cur = bufs[j % 2]
                    cp.wait()
                    if j + 1 < _B_NS:
                        cp = pltpu.async_copy(hx_ref.at[gb2.at[j + 1]],
                                              bufs[(j + 1) % 2],
                                              sems[(j + 1) % 2])

                    def scale(i, c2, cur=cur, j=j):
                        w = plsc.load_gather(
                            wb, [jnp.full((L,), j * _SB + i, jnp.int32)])
                        for t in range(CW // L):
                            ssl = pl.ds(t * L, L)
                            cur[i, ssl] = cur[i, ssl] * w
                        return c2
                    lax.fori_loop(0, _SB, scale, 0)
                    pltpu.sync_copy(cur, acc2.at[db2.at[j]], add=True)
                return c
            lax.fori_loop(0, _B_NB, blk, 0)
            plsc.subcore_barrier()
            pltpu.sync_copy(
                acc2.at[pl.ds(sub * _ROWS_PT, _ROWS_PT)],
                agg_ref.at[pl.ds(chunk * N + sub * _ROWS_PT, _ROWS_PT)])
        pl.when(core == chunk // 2)(_chunk)


_sc_layer2 = functools.partial(
    pl.kernel,
    out_type=jax.ShapeDtypeStruct((HC * N, CW), jnp.float32),
    mesh=plsc.VectorSubcoreMesh(**_MESH),
    compiler_params=pltpu.CompilerParams(needs_layout_passes=False,
                                         use_tc_tiling_on_sc=False),
    scratch_types=[
        pltpu.VMEM_SHARED((N, CW), jnp.float32),
        pltpu.VMEM((_B_EB,), jnp.int32),
        pltpu.VMEM((_B_EB,), jnp.int32),
        pltpu.VMEM((_B_EB,), jnp.int32),
        pltpu.VMEM((_B_EB,), jnp.float32),
        pltpu.VMEM((_B_NS, _SB), jnp.int32),
        pltpu.VMEM((_B_NS, _SB), jnp.int32),
        pltpu.VMEM((_SB, CW), jnp.float32),
        pltpu.VMEM((_SB, CW), jnp.float32),
        pltpu.SemaphoreType.DMA,
        pltpu.SemaphoreType.DMA,
    ],
)(_sc_layer2_body)


def kernel(x, edge_index, edge_type, W1, root1, bias1, W2, root2, bias2):
    src = edge_index[0].astype(jnp.int32)
    dst = edge_index[1].astype(jnp.int32)
    rel = edge_type.astype(jnp.int32)

    winv = _sc_counts(dst, rel)
    sums, wedge = _sc_sums(src, dst, rel, x, winv)

    winv16 = jnp.repeat(winv[:NR].reshape(N, R), DIN, axis=1)
    h1 = _tc_layer1(sums.reshape(N, R * DIN), winv16, x,
                    W1.reshape(R * DIN, H), root1, bias1.reshape(1, H))

    w2x = jnp.concatenate(
        [W2.reshape(R, H, HC, CW), root2.reshape(1, H, HC, CW)],
        axis=0).transpose(2, 0, 1, 3)
    hx = _tc_hall(h1, w2x, bias2.reshape(HC, 1, CW))

    agg = _sc_layer2(src, dst, rel, wedge, hx.reshape(HC * HXROWS, CW))
    return agg.reshape(HC, N, CW).transpose(1, 0, 2).reshape(N, H)


# speedup vs baseline: 4.5631x; 1.1914x over previous
import functools

import jax
import jax.numpy as jnp
from jax import lax
from jax.experimental import pallas as pl
from jax.experimental.pallas import tpu as pltpu
from jax.experimental.pallas import tpu_sc as plsc

N = 10000
R = 8
DIN = 16
H = 512
E = 160000
NR = N * R
NRP = 81920
NC = 2
NS = 16
L = 16
HC = 4
CW = H // HC
HXROWS = (R + 1) * N

_MESH = dict(core_axis_name="c", subcore_axis_name="s",
             num_cores=NC, num_subcores=NS)

_CA_EPT = E // (NC * NS)
_CA_FV = _CA_EPT // L


def _sc_counts_body(dst_ref, rel_ref, part_ref, cnt, db, rb):
    core = lax.axis_index("c")
    sub = lax.axis_index("s")
    wid = sub * NC + core
    e0 = wid * _CA_EPT
    zeros = jnp.zeros((L,), jnp.float32)
    ones = jnp.ones((L,), jnp.float32)
    tail_msk = lax.iota(jnp.int32, L) < (_CA_EPT - _CA_FV * L)

    def zero_body(i, c):
        cnt[pl.ds(i * L, L)] = zeros
        return c
    lax.fori_loop(0, NR // L, zero_body, 0)

    pltpu.sync_copy(dst_ref.at[pl.ds(e0, _CA_EPT)], db.at[pl.ds(0, _CA_EPT)])
    pltpu.sync_copy(rel_ref.at[pl.ds(e0, _CA_EPT)], rb.at[pl.ds(0, _CA_EPT)])
    for i in range(_CA_FV):
        sl = pl.ds(i * L, L)
        plsc.addupdate_scatter(cnt, [db[sl] * R + rb[sl]], ones)
    sl = pl.ds(_CA_FV * L, L)
    seg = jnp.where(tail_msk, db[sl] * R + rb[sl], 0)
    plsc.addupdate_scatter(cnt, [seg], ones, mask=tail_msk)
    pltpu.sync_copy(cnt, part_ref.at[wid])


_sc_counts = functools.partial(
    pl.kernel,
    out_type=jax.ShapeDtypeStruct((NC * NS, NR), jnp.float32),
    mesh=plsc.VectorSubcoreMesh(**_MESH),
    compiler_params=pltpu.CompilerParams(needs_layout_passes=False,
                                         use_tc_tiling_on_sc=False),
    scratch_types=[
        pltpu.VMEM((NR,), jnp.float32),
        pltpu.VMEM((_CA_EPT + L,), jnp.int32),
        pltpu.VMEM((_CA_EPT + L,), jnp.int32),
    ],
)(_sc_counts_body)


def _tc_merge_body(part_ref, out_ref):
    s = jnp.sum(part_ref[...], axis=0)
    out_ref[...] = 1.0 / jnp.maximum(s, 1.0)


def _tc_merge(partials):
    return pl.pallas_call(
        _tc_merge_body,
        out_shape=jax.ShapeDtypeStruct((NR // 128, 128), jnp.float32),
    )(partials)

_EPT = E // NS
_B_EB = 2000
_B_NB = _EPT // _B_EB
_SB = 80
_B_NS = _B_EB // _SB
_VPS = _SB // L
_SUMS_R = NR // NS


def _sc_sums_body(src_ref, dst_ref, rel_ref, x_ref, winv_ref,
                  sums_ref, wedge_ref,
                  acc, sb, db, rb, sb2, ib2, rows, w80, wo, zbuf, sem):
    core = lax.axis_index("c")
    sub = lax.axis_index("s")
    zeros = jnp.zeros((L,), jnp.float32)

    @pl.when(core == 0)
    def _():
        base = sub * _SUMS_R

        def zb(i, c):
            zbuf[i] = zeros
            return c
        lax.fori_loop(0, 500, zb, 0)

        def zdma(j, c):
            pltpu.sync_copy(zbuf, acc.at[pl.ds(base + j * 500, 500)])
            return c
        lax.fori_loop(0, _SUMS_R // 500, zdma, 0)
        plsc.subcore_barrier()

        def blk(b, c):
            e0 = sub * _EPT + b * _B_EB
            pltpu.sync_copy(src_ref.at[pl.ds(e0, _B_EB)], sb)
            pltpu.sync_copy(dst_ref.at[pl.ds(e0, _B_EB)], db)
            pltpu.sync_copy(rel_ref.at[pl.ds(e0, _B_EB)], rb)
            for i in range(_B_EB // L):
                sl = pl.ds(i * L, L)
                dsl = pl.ds((i % _VPS) * L, L)
                ib2[i // _VPS, dsl] = db[sl] * R + rb[sl]
                sb2[i // _VPS, dsl] = sb[sl]
            for j in range(_B_NS):
                pltpu.async_copy(x_ref.at[sb2.at[j]], rows, sem).wait()
                pltpu.sync_copy(rows, acc.at[ib2.at[j]], add=True)
            return c
        lax.fori_loop(0, _B_NB, blk, 0)
        plsc.subcore_barrier()
        pltpu.sync_copy(acc.at[pl.ds(base, _SUMS_R)],
                        sums_ref.at[pl.ds(base, _SUMS_R)])

    @pl.when(core == 1)
    def _():
        def blk(b, c):
            e0 = sub * _EPT + b * _B_EB
            pltpu.sync_copy(dst_ref.at[pl.ds(e0, _B_EB)], db)
            pltpu.sync_copy(rel_ref.at[pl.ds(e0, _B_EB)], rb)
            for i in range(_B_EB // L):
                sl = pl.ds(i * L, L)
                dsl = pl.ds((i % _VPS) * L, L)
                ib2[i // _VPS, dsl] = db[sl] * R + rb[sl]
            for j in range(_B_NS):
                pltpu.async_copy(winv_ref.at[ib2.at[j]], w80, sem).wait()
                for t in range(_VPS):
                    wo[pl.ds(j * _SB + t * L, L)] = w80[pl.ds(t * L, L)]
            pltpu.sync_copy(wo, wedge_ref.at[pl.ds(e0, _B_EB)])
            return c
        lax.fori_loop(0, _B_NB, blk, 0)


_sc_sums = functools.partial(
    pl.kernel,
    out_type=(jax.ShapeDtypeStruct((NR, DIN), jnp.float32),
              jax.ShapeDtypeStruct((E,), jnp.float32)),
    mesh=plsc.VectorSubcoreMesh(**_MESH),
    compiler_params=pltpu.CompilerParams(needs_layout_passes=False,
                                         use_tc_tiling_on_sc=False),
    scratch_types=[
        pltpu.VMEM_SHARED((NR, DIN), jnp.float32),
        pltpu.VMEM((_B_EB,), jnp.int32),
        pltpu.VMEM((_B_EB,), jnp.int32),
        pltpu.VMEM((_B_EB,), jnp.int32),
        pltpu.VMEM((_B_NS, _SB), jnp.int32),
        pltpu.VMEM((_B_NS, _SB), jnp.int32),
        pltpu.VMEM((_SB, DIN), jnp.float32),
        pltpu.VMEM((_SB,), jnp.float32),
        pltpu.VMEM((_B_EB,), jnp.float32),
        pltpu.VMEM((500, DIN), jnp.float32),
        pltpu.SemaphoreType.DMA,
    ],
)(_sc_sums_body)

_BN = 1000


def _tc1_body(sums_ref, winv_ref, x_ref, w1_ref, r1_ref, b1_ref, out_ref):
    means = sums_ref[...] * winv_ref[...]
    h = jnp.dot(means, w1_ref[...], preferred_element_type=jnp.float32)
    h = h + jnp.dot(x_ref[...], r1_ref[...], preferred_element_type=jnp.float32)
    h = h + b1_ref[...]
    out_ref[...] = jnp.maximum(h, 0.0)


def _tc_layer1(sums2d, winv16, x, w1f, root1, bias1):
    return pl.pallas_call(
        _tc1_body,
        grid=(N // _BN,),
        in_specs=[
            pl.BlockSpec((_BN, R * DIN), lambda i: (i, 0)),
            pl.BlockSpec((_BN, R * DIN), lambda i: (i, 0)),
            pl.BlockSpec((_BN, DIN), lambda i: (i, 0)),
            pl.BlockSpec((R * DIN, H), lambda i: (0, 0)),
            pl.BlockSpec((DIN, H), lambda i: (0, 0)),
            pl.BlockSpec((1, H), lambda i: (0, 0)),
        ],
        out_specs=pl.BlockSpec((_BN, H), lambda i: (i, 0)),
        out_shape=jax.ShapeDtypeStruct((N, H), jnp.float32),
    )(sums2d, winv16, x, w1f, root1, bias1)


def _tc2_body(h1_ref, w_ref, b2_ref, out_ref):
    r = pl.program_id(2)
    v = jnp.dot(h1_ref[...], w_ref[...].reshape(H, CW),
                preferred_element_type=jnp.float32)
    out_ref[...] = v + jnp.where(r == R, b2_ref[0], 0.0)


def _tc_hall(h1, w2x, bias2c):
    return pl.pallas_call(
        _tc2_body,
        grid=(N // _BN, HC, R + 1),
        in_specs=[
            pl.BlockSpec((_BN, H), lambda nb, c, r: (nb, 0)),
            pl.BlockSpec((1, 1, H, CW), lambda nb, c, r: (c, r, 0, 0)),
            pl.BlockSpec((1, 1, CW), lambda nb, c, r: (c, 0, 0)),
        ],
        out_specs=pl.BlockSpec(
            (_BN, CW),
            lambda nb, c, r: (c * (HXROWS // _BN) + r * (N // _BN) + nb, 0)),
        out_shape=jax.ShapeDtypeStruct((HC * HXROWS, CW), jnp.float32),
    )(h1, w2x, bias2c)

_ROWS_PT = N // NS


def _sc_layer2_body(src_ref, dst_ref, rel_ref, we_ref, hx_ref, agg_ref,
                    acc2, sb, db, rb, wb, gb2, db2, rows_a, rows_b,
                    sem_a, sem_b):
    core = lax.axis_index("c")
    sub = lax.axis_index("s")

    for chunk in range(HC):
        def _chunk(chunk=chunk):
            pltpu.sync_copy(
                hx_ref.at[pl.ds(chunk * HXROWS + R * N + sub * _ROWS_PT,
                                _ROWS_PT)],
                acc2.at[pl.ds(sub * _ROWS_PT, _ROWS_PT)])
            plsc.subcore_barrier()

            def blk(b, c):
                e0 = sub * _EPT + b * _B_EB
                pltpu.sync_copy(src_ref.at[pl.ds(e0, _B_EB)], sb)
                pltpu.sync_copy(dst_ref.at[pl.ds(e0, _B_EB)], db)
                pltpu.sync_copy(rel_ref.at[pl.ds(e0, _B_EB)], rb)
                pltpu.sync_copy(we_ref.at[pl.ds(e0, _B_EB)], wb)
                for i in range(_B_EB // L):
                    sl = pl.ds(i * L, L)
                    dsl = pl.ds((i % _VPS) * L, L)
                    gb2[i // _VPS, dsl] = rb[sl] * N + sb[sl] + chunk * HXROWS
                    db2[i // _VPS, dsl] = db[sl]
                bufs = (rows_a, rows_b)
                sems = (sem_a, sem_b)
                cp = pltpu.async_copy(hx_ref.at[gb2.at[0]], rows_a, sem_a)
                for j in range(_B_NS):
                    cur = bufs[j % 2]
                    cp.wait()
                    if j + 1 < _B_NS:
                        cp = pltpu.async_copy(hx_ref.at[gb2.at[j + 1]],
                                              bufs[(j + 1) % 2],
                                              sems[(j + 1) % 2])

                    def scale(i, c2, cur=cur, j=j):
                        w = plsc.load_gather(
                            wb, [jnp.full((L,), j * _SB + i, jnp.int32)])
                        for t in range(CW // L):
                            ssl = pl.ds(t * L, L)
                            cur[i, ssl] = cur[i, ssl] * w
                        return c2
                    lax.fori_loop(0, _SB, scale, 0)
                    pltpu.sync_copy(cur, acc2.at[db2.at[j]], add=True)
                return c
            lax.fori_loop(0, _B_NB, blk, 0)
            plsc.subcore_barrier()
            pltpu.sync_copy(
                acc2.at[pl.ds(sub * _ROWS_PT, _ROWS_PT)],
                agg_ref.at[pl.ds(sub * _ROWS_PT, _ROWS_PT),
                           pl.ds(chunk * CW, CW)])
        pl.when(core == chunk // 2)(_chunk)


_sc_layer2 = functools.partial(
    pl.kernel,
    out_type=jax.ShapeDtypeStruct((N, H), jnp.float32),
    mesh=plsc.VectorSubcoreMesh(**_MESH),
    compiler_params=pltpu.CompilerParams(needs_layout_passes=False,
                                         use_tc_tiling_on_sc=False),
    scratch_types=[
        pltpu.VMEM_SHARED((N, CW), jnp.float32),
        pltpu.VMEM((_B_EB,), jnp.int32),
        pltpu.VMEM((_B_EB,), jnp.int32),
        pltpu.VMEM((_B_EB,), jnp.int32),
        pltpu.VMEM((_B_EB,), jnp.float32),
        pltpu.VMEM((_B_NS, _SB), jnp.int32),
        pltpu.VMEM((_B_NS, _SB), jnp.int32),
        pltpu.VMEM((_SB, CW), jnp.float32),
        pltpu.VMEM((_SB, CW), jnp.float32),
        pltpu.SemaphoreType.DMA,
        pltpu.SemaphoreType.DMA,
    ],
)(_sc_layer2_body)


def kernel(x, edge_index, edge_type, W1, root1, bias1, W2, root2, bias2):
    src = edge_index[0].astype(jnp.int32)
    dst = edge_index[1].astype(jnp.int32)
    rel = edge_type.astype(jnp.int32)

    parts = _sc_counts(dst, rel)
    winv2d = _tc_merge(parts.reshape(NC * NS, NR // 128, 128))
    winv_flat = winv2d.reshape(NR)
    sums, wedge = _sc_sums(src, dst, rel, x, winv_flat)

    winv16 = jnp.repeat(winv2d.reshape(N, R), DIN, axis=1)
    h1 = _tc_layer1(sums.reshape(N, R * DIN), winv16, x,
                    W1.reshape(R * DIN, H), root1, bias1.reshape(1, H))

    w2x = jnp.concatenate(
        [W2.reshape(R, H, HC, CW), root2.reshape(1, H, HC, CW)],
        axis=0).transpose(2, 0, 1, 3)
    hx = _tc_hall(h1, w2x, bias2.reshape(HC, 1, CW))

    return _sc_layer2(src, dst, rel, wedge, hx)


# speedup vs baseline: 4.7184x; 1.0340x over previous
import functools

import jax
import jax.numpy as jnp
from jax import lax
from jax.experimental import pallas as pl
from jax.experimental.pallas import tpu as pltpu
from jax.experimental.pallas import tpu_sc as plsc

N = 10000
R = 8
DIN = 16
H = 512
E = 160000
NR = N * R
NRP = 81920
NC = 2
NS = 16
L = 16
HC = 4
CW = H // HC
HXROWS = (R + 1) * N

_MESH = dict(core_axis_name="c", subcore_axis_name="s",
             num_cores=NC, num_subcores=NS)

_CA_EPT = E // (NC * NS)
_CA_FV = _CA_EPT // L


def _sc_counts_body(dst_ref, rel_ref, part_ref, cnt, db, rb):
    core = lax.axis_index("c")
    sub = lax.axis_index("s")
    wid = sub * NC + core
    e0 = wid * _CA_EPT
    zeros = jnp.zeros((L,), jnp.float32)
    ones = jnp.ones((L,), jnp.float32)
    tail_msk = lax.iota(jnp.int32, L) < (_CA_EPT - _CA_FV * L)

    def zero_body(i, c):
        cnt[pl.ds(i * L, L)] = zeros
        return c
    lax.fori_loop(0, NR // L, zero_body, 0)

    pltpu.sync_copy(dst_ref.at[pl.ds(e0, _CA_EPT)], db.at[pl.ds(0, _CA_EPT)])
    pltpu.sync_copy(rel_ref.at[pl.ds(e0, _CA_EPT)], rb.at[pl.ds(0, _CA_EPT)])
    for i in range(_CA_FV):
        sl = pl.ds(i * L, L)
        plsc.addupdate_scatter(cnt, [db[sl] * R + rb[sl]], ones)
    sl = pl.ds(_CA_FV * L, L)
    seg = jnp.where(tail_msk, db[sl] * R + rb[sl], 0)
    plsc.addupdate_scatter(cnt, [seg], ones, mask=tail_msk)
    pltpu.sync_copy(cnt, part_ref.at[wid])


_sc_counts = functools.partial(
    pl.kernel,
    out_type=jax.ShapeDtypeStruct((NC * NS, NR), jnp.float32),
    mesh=plsc.VectorSubcoreMesh(**_MESH),
    compiler_params=pltpu.CompilerParams(needs_layout_passes=False,
                                         use_tc_tiling_on_sc=False),
    scratch_types=[
        pltpu.VMEM((NR,), jnp.float32),
        pltpu.VMEM((_CA_EPT + L,), jnp.int32),
        pltpu.VMEM((_CA_EPT + L,), jnp.int32),
    ],
)(_sc_counts_body)


def _tc_merge_body(part_ref, out_ref):
    s = jnp.sum(part_ref[...], axis=0)
    out_ref[...] = 1.0 / jnp.maximum(s, 1.0)


def _tc_merge(partials):
    return pl.pallas_call(
        _tc_merge_body,
        out_shape=jax.ShapeDtypeStruct((NR // 128, 128), jnp.float32),
    )(partials)

_EPT = E // NS
_B_EB = 2000
_B_NB = _EPT // _B_EB
_SB = 80
_B_NS = _B_EB // _SB
_VPS = _SB // L
_SUMS_R = NR // NS


def _sc_sums_body(src_ref, dst_ref, rel_ref, x_ref, winv_ref,
                  sums_ref, wedge_ref,
                  acc, sb, db, rb, sb2, ib2, rows, w80, wo, zbuf, sem):
    core = lax.axis_index("c")
    sub = lax.axis_index("s")
    zeros = jnp.zeros((L,), jnp.float32)

    @pl.when(core == 0)
    def _():
        base = sub * _SUMS_R

        def zb(i, c):
            zbuf[i] = zeros
            return c
        lax.fori_loop(0, 500, zb, 0)

        def zdma(j, c):
            pltpu.sync_copy(zbuf, acc.at[pl.ds(base + j * 500, 500)])
            return c
        lax.fori_loop(0, _SUMS_R // 500, zdma, 0)
        plsc.subcore_barrier()

        def blk(b, c):
            e0 = sub * _EPT + b * _B_EB
            pltpu.sync_copy(src_ref.at[pl.ds(e0, _B_EB)], sb)
            pltpu.sync_copy(dst_ref.at[pl.ds(e0, _B_EB)], db)
            pltpu.sync_copy(rel_ref.at[pl.ds(e0, _B_EB)], rb)
            for i in range(_B_EB // L):
                sl = pl.ds(i * L, L)
                dsl = pl.ds((i % _VPS) * L, L)
                ib2[i // _VPS, dsl] = db[sl] * R + rb[sl]
                sb2[i // _VPS, dsl] = sb[sl]
            for j in range(_B_NS):
                pltpu.async_copy(x_ref.at[sb2.at[j]], rows, sem).wait()
                pltpu.sync_copy(rows, acc.at[ib2.at[j]], add=True)
            return c
        lax.fori_loop(0, _B_NB, blk, 0)
        plsc.subcore_barrier()
        pltpu.sync_copy(acc.at[pl.ds(base, _SUMS_R)],
                        sums_ref.at[pl.ds(base, _SUMS_R)])

    @pl.when(core == 1)
    def _():
        def blk(b, c):
            e0 = sub * _EPT + b * _B_EB
            pltpu.sync_copy(dst_ref.at[pl.ds(e0, _B_EB)], db)
            pltpu.sync_copy(rel_ref.at[pl.ds(e0, _B_EB)], rb)
            for i in range(_B_EB // L):
                sl = pl.ds(i * L, L)
                dsl = pl.ds((i % _VPS) * L, L)
                ib2[i // _VPS, dsl] = db[sl] * R + rb[sl]
            for j in range(_B_NS):
                pltpu.async_copy(winv_ref.at[ib2.at[j]], w80, sem).wait()
                for t in range(_VPS):
                    wo[pl.ds(j * _SB + t * L, L)] = w80[pl.ds(t * L, L)]
            pltpu.sync_copy(wo, wedge_ref.at[pl.ds(e0, _B_EB)])
            return c
        lax.fori_loop(0, _B_NB, blk, 0)


_sc_sums = functools.partial(
    pl.kernel,
    out_type=(jax.ShapeDtypeStruct((NR, DIN), jnp.float32),
              jax.ShapeDtypeStruct((E,), jnp.float32)),
    mesh=plsc.VectorSubcoreMesh(**_MESH),
    compiler_params=pltpu.CompilerParams(needs_layout_passes=False,
                                         use_tc_tiling_on_sc=False),
    scratch_types=[
        pltpu.VMEM_SHARED((NR, DIN), jnp.float32),
        pltpu.VMEM((_B_EB,), jnp.int32),
        pltpu.VMEM((_B_EB,), jnp.int32),
        pltpu.VMEM((_B_EB,), jnp.int32),
        pltpu.VMEM((_B_NS, _SB), jnp.int32),
        pltpu.VMEM((_B_NS, _SB), jnp.int32),
        pltpu.VMEM((_SB, DIN), jnp.float32),
        pltpu.VMEM((_SB,), jnp.float32),
        pltpu.VMEM((_B_EB,), jnp.float32),
        pltpu.VMEM((500, DIN), jnp.float32),
        pltpu.SemaphoreType.DMA,
    ],
)(_sc_sums_body)

_BN = 1000


def _tc1_body(sums_ref, winv_ref, x_ref, w1_ref, r1_ref, b1_ref, out_ref):
    means = sums_ref[...] * winv_ref[...]
    h = jnp.dot(means, w1_ref[...], preferred_element_type=jnp.float32)
    h = h + jnp.dot(x_ref[...], r1_ref[...], preferred_element_type=jnp.float32)
    h = h + b1_ref[...]
    out_ref[...] = jnp.maximum(h, 0.0).astype(jnp.bfloat16)


def _tc_layer1(sums2d, winv16, x, w1f, root1, bias1):
    return pl.pallas_call(
        _tc1_body,
        grid=(N // _BN,),
        in_specs=[
            pl.BlockSpec((_BN, R * DIN), lambda i: (i, 0)),
            pl.BlockSpec((_BN, R * DIN), lambda i: (i, 0)),
            pl.BlockSpec((_BN, DIN), lambda i: (i, 0)),
            pl.BlockSpec((R * DIN, H), lambda i: (0, 0)),
            pl.BlockSpec((DIN, H), lambda i: (0, 0)),
            pl.BlockSpec((1, H), lambda i: (0, 0)),
        ],
        out_specs=pl.BlockSpec((_BN, H), lambda i: (i, 0)),
        out_shape=jax.ShapeDtypeStruct((N, H), jnp.bfloat16),
    )(sums2d, winv16, x, w1f, root1, bias1)


def _tc2_body(h1_ref, w_ref, b2_ref, out_ref):
    r = pl.program_id(2)
    v = jnp.dot(h1_ref[...], w_ref[...].reshape(H, CW),
                preferred_element_type=jnp.float32)
    out_ref[...] = v + jnp.where(r == R, b2_ref[0], 0.0)


def _tc_hall(h1, w2x, bias2c):
    return pl.pallas_call(
        _tc2_body,
        grid=(N // _BN, HC, R + 1),
        in_specs=[
            pl.BlockSpec((_BN, H), lambda nb, c, r: (nb, 0)),
            pl.BlockSpec((1, 1, H, CW), lambda nb, c, r: (c, r, 0, 0)),
            pl.BlockSpec((1, 1, CW), lambda nb, c, r: (c, 0, 0)),
        ],
        out_specs=pl.BlockSpec(
            (_BN, CW),
            lambda nb, c, r: (c * (HXROWS // _BN) + r * (N // _BN) + nb, 0)),
        out_shape=jax.ShapeDtypeStruct((HC * HXROWS, CW), jnp.float32),
    )(h1, w2x, bias2c)

_ROWS_PT = N // NS


def _sc_layer2_body(src_ref, dst_ref, rel_ref, we_ref, hx_ref, agg_ref,
                    acc2, sb, db, rb, wb, gb2, db2, rows_a, rows_b,
                    sem_a, sem_b):
    core = lax.axis_index("c")
    sub = lax.axis_index("s")

    for chunk in range(HC):
        def _chunk(chunk=chunk):
            pltpu.sync_copy(
                hx_ref.at[pl.ds(chunk * HXROWS + R * N + sub * _ROWS_PT,
                                _ROWS_PT)],
                acc2.at[pl.ds(sub * _ROWS_PT, _ROWS_PT)])
            plsc.subcore_barrier()

            def blk(b, c):
                e0 = sub * _EPT + b * _B_EB
                pltpu.sync_copy(src_ref.at[pl.ds(e0, _B_EB)], sb)
                pltpu.sync_copy(dst_ref.at[pl.ds(e0, _B_EB)], db)
                pltpu.sync_copy(rel_ref.at[pl.ds(e0, _B_EB)], rb)
                pltpu.sync_copy(we_ref.at[pl.ds(e0, _B_EB)], wb)
                for i in range(_B_EB // L):
                    sl = pl.ds(i * L, L)
                    dsl = pl.ds((i % _VPS) * L, L)
                    gb2[i // _VPS, dsl] = rb[sl] * N + sb[sl] + chunk * HXROWS
                    db2[i // _VPS, dsl] = db[sl]
                bufs = (rows_a, rows_b)
                sems = (sem_a, sem_b)
                cp = pltpu.async_copy(hx_ref.at[gb2.at[0]], rows_a, sem_a)
                for j in range(_B_NS):
                    cur = bufs[j % 2]
                    cp.wait()
                    if j + 1 < _B_NS:
                        cp = pltpu.async_copy(hx_ref.at[gb2.at[j + 1]],
                                              bufs[(j + 1) % 2],
                                              sems[(j + 1) % 2])

                    def scale(i, c2, cur=cur, j=j):
                        w = plsc.load_gather(
                            wb, [jnp.full((L,), j * _SB + i, jnp.int32)])
                        for t in range(CW // L):
                            ssl = pl.ds(t * L, L)
                            cur[i, ssl] = cur[i, ssl] * w
                        return c2
                    lax.fori_loop(0, _SB, scale, 0)
                    pltpu.sync_copy(cur, acc2.at[db2.at[j]], add=True)
                return c
            lax.fori_loop(0, _B_NB, blk, 0)
            plsc.subcore_barrier()
            pltpu.sync_copy(
                acc2.at[pl.ds(sub * _ROWS_PT, _ROWS_PT)],
                agg_ref.at[pl.ds(sub * _ROWS_PT, _ROWS_PT),
                           pl.ds(chunk * CW, CW)])
        pl.when(core == chunk // 2)(_chunk)


_sc_layer2 = functools.partial(
    pl.kernel,
    out_type=jax.ShapeDtypeStruct((N, H), jnp.float32),
    mesh=plsc.VectorSubcoreMesh(**_MESH),
    compiler_params=pltpu.CompilerParams(needs_layout_passes=False,
                                         use_tc_tiling_on_sc=False),
    scratch_types=[
        pltpu.VMEM_SHARED((N, CW), jnp.float32),
        pltpu.VMEM((_B_EB,), jnp.int32),
        pltpu.VMEM((_B_EB,), jnp.int32),
        pltpu.VMEM((_B_EB,), jnp.int32),
        pltpu.VMEM((_B_EB,), jnp.float32),
        pltpu.VMEM((_B_NS, _SB), jnp.int32),
        pltpu.VMEM((_B_NS, _SB), jnp.int32),
        pltpu.VMEM((_SB, CW), jnp.float32),
        pltpu.VMEM((_SB, CW), jnp.float32),
        pltpu.SemaphoreType.DMA,
        pltpu.SemaphoreType.DMA,
    ],
)(_sc_layer2_body)


def kernel(x, edge_index, edge_type, W1, root1, bias1, W2, root2, bias2):
    src = edge_index[0].astype(jnp.int32)
    dst = edge_index[1].astype(jnp.int32)
    rel = edge_type.astype(jnp.int32)

    parts = _sc_counts(dst, rel)
    winv2d = _tc_merge(parts.reshape(NC * NS, NR // 128, 128))
    winv_flat = winv2d.reshape(NR)
    sums, wedge = _sc_sums(src, dst, rel, x, winv_flat)

    winv16 = jnp.repeat(winv2d.reshape(N, R), DIN, axis=1)
    h1 = _tc_layer1(sums.reshape(N, R * DIN), winv16, x,
                    W1.reshape(R * DIN, H), root1, bias1.reshape(1, H))

    w2x = jnp.concatenate(
        [W2.reshape(R, H, HC, CW), root2.reshape(1, H, HC, CW)],
        axis=0).transpose(2, 0, 1, 3).astype(jnp.bfloat16)
    hx = _tc_hall(h1, w2x, bias2.reshape(HC, 1, CW))

    return _sc_layer2(src, dst, rel, wedge, hx)


# speedup vs baseline: 4.7206x; 1.0005x over previous
import functools

import jax
import jax.numpy as jnp
from jax import lax
from jax.experimental import pallas as pl
from jax.experimental.pallas import tpu as pltpu
from jax.experimental.pallas import tpu_sc as plsc

N = 10000
R = 8
DIN = 16
H = 512
E = 160000
NR = N * R
NRP = 81920
NC = 2
NS = 16
L = 16
HC = 4
CW = H // HC
HXROWS = (R + 1) * N

_MESH = dict(core_axis_name="c", subcore_axis_name="s",
             num_cores=NC, num_subcores=NS)

_CA_EPT = E // (NC * NS)
_CA_FV = _CA_EPT // L


def _sc_counts_body(dst_ref, rel_ref, part_ref, cnt, db, rb):
    core = lax.axis_index("c")
    sub = lax.axis_index("s")
    wid = sub * NC + core
    e0 = wid * _CA_EPT
    zeros = jnp.zeros((L,), jnp.float32)
    ones = jnp.ones((L,), jnp.float32)
    tail_msk = lax.iota(jnp.int32, L) < (_CA_EPT - _CA_FV * L)

    def zero_body(i, c):
        cnt[pl.ds(i * L, L)] = zeros
        return c
    lax.fori_loop(0, NR // L, zero_body, 0)

    pltpu.sync_copy(dst_ref.at[pl.ds(e0, _CA_EPT)], db.at[pl.ds(0, _CA_EPT)])
    pltpu.sync_copy(rel_ref.at[pl.ds(e0, _CA_EPT)], rb.at[pl.ds(0, _CA_EPT)])
    for i in range(_CA_FV):
        sl = pl.ds(i * L, L)
        plsc.addupdate_scatter(cnt, [db[sl] * R + rb[sl]], ones)
    sl = pl.ds(_CA_FV * L, L)
    seg = jnp.where(tail_msk, db[sl] * R + rb[sl], 0)
    plsc.addupdate_scatter(cnt, [seg], ones, mask=tail_msk)
    pltpu.sync_copy(cnt, part_ref.at[wid])


_sc_counts = functools.partial(
    pl.kernel,
    out_type=jax.ShapeDtypeStruct((NC * NS, NR), jnp.float32),
    mesh=plsc.VectorSubcoreMesh(**_MESH),
    compiler_params=pltpu.CompilerParams(needs_layout_passes=False,
                                         use_tc_tiling_on_sc=False),
    scratch_types=[
        pltpu.VMEM((NR,), jnp.float32),
        pltpu.VMEM((_CA_EPT + L,), jnp.int32),
        pltpu.VMEM((_CA_EPT + L,), jnp.int32),
    ],
)(_sc_counts_body)


def _tc_merge_body(part_ref, out_ref):
    s = jnp.sum(part_ref[...], axis=0)
    out_ref[...] = 1.0 / jnp.maximum(s, 1.0)


def _tc_merge(partials):
    return pl.pallas_call(
        _tc_merge_body,
        out_shape=jax.ShapeDtypeStruct((NR // 128, 128), jnp.float32),
    )(partials)

_EPT = E // NS
_B_EB = 2000
_B_NB = _EPT // _B_EB
_SB = 80
_B_NS = _B_EB // _SB
_VPS = _SB // L
_SUMS_R = NR // NS


def _sc_sums_body(src_ref, dst_ref, rel_ref, x_ref, winv_ref,
                  sums_ref, wedge_ref,
                  acc, sb, db, rb, sb2, ib2, rows, w80, wo, zbuf, sem):
    core = lax.axis_index("c")
    sub = lax.axis_index("s")
    zeros = jnp.zeros((L,), jnp.float32)

    @pl.when(core == 0)
    def _():
        base = sub * _SUMS_R

        def zb(i, c):
            zbuf[i] = zeros
            return c
        lax.fori_loop(0, 500, zb, 0)

        def zdma(j, c):
            pltpu.sync_copy(zbuf, acc.at[pl.ds(base + j * 500, 500)])
            return c
        lax.fori_loop(0, _SUMS_R // 500, zdma, 0)
        plsc.subcore_barrier()

        def blk(b, c):
            e0 = sub * _EPT + b * _B_EB
            pltpu.sync_copy(src_ref.at[pl.ds(e0, _B_EB)], sb)
            pltpu.sync_copy(dst_ref.at[pl.ds(e0, _B_EB)], db)
            pltpu.sync_copy(rel_ref.at[pl.ds(e0, _B_EB)], rb)
            for i in range(_B_EB // L):
                sl = pl.ds(i * L, L)
                dsl = pl.ds((i % _VPS) * L, L)
                ib2[i // _VPS, dsl] = db[sl] * R + rb[sl]
                sb2[i // _VPS, dsl] = sb[sl]
            for j in range(_B_NS):
                pltpu.async_copy(x_ref.at[sb2.at[j]], rows, sem).wait()
                pltpu.sync_copy(rows, acc.at[ib2.at[j]], add=True)
            return c
        lax.fori_loop(0, _B_NB, blk, 0)
        plsc.subcore_barrier()
        pltpu.sync_copy(acc.at[pl.ds(base, _SUMS_R)],
                        sums_ref.at[pl.ds(base, _SUMS_R)])

    @pl.when(core == 1)
    def _():
        def blk(b, c):
            e0 = sub * _EPT + b * _B_EB
            pltpu.sync_copy(dst_ref.at[pl.ds(e0, _B_EB)], db)
            pltpu.sync_copy(rel_ref.at[pl.ds(e0, _B_EB)], rb)
            for i in range(_B_EB // L):
                sl = pl.ds(i * L, L)
                dsl = pl.ds((i % _VPS) * L, L)
                ib2[i // _VPS, dsl] = db[sl] * R + rb[sl]
            for j in range(_B_NS):
                pltpu.async_copy(winv_ref.at[ib2.at[j]], w80, sem).wait()
                for t in range(_VPS):
                    wo[pl.ds(j * _SB + t * L, L)] = w80[pl.ds(t * L, L)]
            pltpu.sync_copy(wo, wedge_ref.at[pl.ds(e0, _B_EB)])
            return c
        lax.fori_loop(0, _B_NB, blk, 0)


_sc_sums = functools.partial(
    pl.kernel,
    out_type=(jax.ShapeDtypeStruct((NR, DIN), jnp.float32),
              jax.ShapeDtypeStruct((E,), jnp.float32)),
    mesh=plsc.VectorSubcoreMesh(**_MESH),
    compiler_params=pltpu.CompilerParams(needs_layout_passes=False,
                                         use_tc_tiling_on_sc=False),
    scratch_types=[
        pltpu.VMEM_SHARED((NR, DIN), jnp.float32),
        pltpu.VMEM((_B_EB,), jnp.int32),
        pltpu.VMEM((_B_EB,), jnp.int32),
        pltpu.VMEM((_B_EB,), jnp.int32),
        pltpu.VMEM((_B_NS, _SB), jnp.int32),
        pltpu.VMEM((_B_NS, _SB), jnp.int32),
        pltpu.VMEM((_SB, DIN), jnp.float32),
        pltpu.VMEM((_SB,), jnp.float32),
        pltpu.VMEM((_B_EB,), jnp.float32),
        pltpu.VMEM((500, DIN), jnp.float32),
        pltpu.SemaphoreType.DMA,
    ],
)(_sc_sums_body)

_BN = 1000


def _tc1_body(sums_ref, winv_ref, x_ref, w1_ref, r1_ref, b1_ref, out_ref):
    means = sums_ref[...] * winv_ref[...]
    h = jnp.dot(means, w1_ref[...], preferred_element_type=jnp.float32)
    h = h + jnp.dot(x_ref[...], r1_ref[...], preferred_element_type=jnp.float32)
    h = h + b1_ref[...]
    out_ref[...] = jnp.maximum(h, 0.0).astype(jnp.bfloat16)


def _tc_layer1(sums2d, winv16, x, w1f, root1, bias1):
    return pl.pallas_call(
        _tc1_body,
        grid=(N // _BN,),
        in_specs=[
            pl.BlockSpec((_BN, R * DIN), lambda i: (i, 0)),
            pl.BlockSpec((_BN, R * DIN), lambda i: (i, 0)),
            pl.BlockSpec((_BN, DIN), lambda i: (i, 0)),
            pl.BlockSpec((R * DIN, H), lambda i: (0, 0)),
            pl.BlockSpec((DIN, H), lambda i: (0, 0)),
            pl.BlockSpec((1, H), lambda i: (0, 0)),
        ],
        out_specs=pl.BlockSpec((_BN, H), lambda i: (i, 0)),
        out_shape=jax.ShapeDtypeStruct((N, H), jnp.bfloat16),
    )(sums2d, winv16, x, w1f, root1, bias1)


def _tc2_body(h1_ref, w_ref, b2_ref, out_ref):
    r = pl.program_id(2)
    v = jnp.dot(h1_ref[...], w_ref[...].reshape(H, CW),
                preferred_element_type=jnp.float32)
    out_ref[...] = v + jnp.where(r == R, b2_ref[0], 0.0)


def _tc_hall(h1, w2x, bias2c):
    return pl.pallas_call(
        _tc2_body,
        grid=(N // _BN, HC, R + 1),
        in_specs=[
            pl.BlockSpec((_BN, H), lambda nb, c, r: (nb, 0)),
            pl.BlockSpec((1, 1, H, CW), lambda nb, c, r: (c, r, 0, 0)),
            pl.BlockSpec((1, 1, CW), lambda nb, c, r: (c, 0, 0)),
        ],
        out_specs=pl.BlockSpec(
            (_BN, CW),
            lambda nb, c, r: (c * (HXROWS // _BN) + r * (N // _BN) + nb, 0)),
        out_shape=jax.ShapeDtypeStruct((HC * HXROWS, CW), jnp.float32),
    )(h1, w2x, bias2c)

_ROWS_PT = N // NS


def _sc_layer2_body(src_ref, dst_ref, rel_ref, we_ref, hx_ref, agg_ref,
                    acc2, sb, db, rb, wb, gb2, db2, rows_a, rows_b,
                    sem_a, sem_b, sem_sa, sem_sb):
    core = lax.axis_index("c")
    sub = lax.axis_index("s")

    for chunk in range(HC):
        def _chunk(chunk=chunk):
            pltpu.sync_copy(
                hx_ref.at[pl.ds(chunk * HXROWS + R * N + sub * _ROWS_PT,
                                _ROWS_PT)],
                acc2.at[pl.ds(sub * _ROWS_PT, _ROWS_PT)])
            plsc.subcore_barrier()

            def blk(b, c):
                e0 = sub * _EPT + b * _B_EB
                pltpu.sync_copy(src_ref.at[pl.ds(e0, _B_EB)], sb)
                pltpu.sync_copy(dst_ref.at[pl.ds(e0, _B_EB)], db)
                pltpu.sync_copy(rel_ref.at[pl.ds(e0, _B_EB)], rb)
                pltpu.sync_copy(we_ref.at[pl.ds(e0, _B_EB)], wb)
                for i in range(_B_EB // L):
                    sl = pl.ds(i * L, L)
                    dsl = pl.ds((i % _VPS) * L, L)
                    gb2[i // _VPS, dsl] = rb[sl] * N + sb[sl] + chunk * HXROWS
                    db2[i // _VPS, dsl] = db[sl]
                bufs = (rows_a, rows_b)
                gsems = (sem_a, sem_b)
                ssems = (sem_sa, sem_sb)
                pend = [None, None]
                cp = pltpu.async_copy(hx_ref.at[gb2.at[0]], rows_a, sem_a)
                for j in range(_B_NS):
                    b = j % 2
                    cur = bufs[b]
                    cp.wait()
                    if j + 1 < _B_NS:
                        o = (j + 1) % 2
                        if pend[o] is not None:
                            pend[o].wait()
                            pend[o] = None
                        cp = pltpu.async_copy(hx_ref.at[gb2.at[j + 1]],
                                              bufs[o], gsems[o])

                    def scale(i, c2, cur=cur, j=j):
                        w = plsc.load_gather(
                            wb, [jnp.full((L,), j * _SB + i, jnp.int32)])
                        for t in range(CW // L):
                            ssl = pl.ds(t * L, L)
                            cur[i, ssl] = cur[i, ssl] * w
                        return c2
                    lax.fori_loop(0, _SB, scale, 0)
                    pend[b] = pltpu.async_copy(cur, acc2.at[db2.at[j]],
                                               ssems[b], add=True)
                for p in pend:
                    if p is not None:
                        p.wait()
                return c
            lax.fori_loop(0, _B_NB, blk, 0)
            plsc.subcore_barrier()
            pltpu.sync_copy(
                acc2.at[pl.ds(sub * _ROWS_PT, _ROWS_PT)],
                agg_ref.at[pl.ds(sub * _ROWS_PT, _ROWS_PT),
                           pl.ds(chunk * CW, CW)])
        pl.when(core == chunk // 2)(_chunk)


_sc_layer2 = functools.partial(
    pl.kernel,
    out_type=jax.ShapeDtypeStruct((N, H), jnp.float32),
    mesh=plsc.VectorSubcoreMesh(**_MESH),
    compiler_params=pltpu.CompilerParams(needs_layout_passes=False,
                                         use_tc_tiling_on_sc=False),
    scratch_types=[
        pltpu.VMEM_SHARED((N, CW), jnp.float32),
        pltpu.VMEM((_B_EB,), jnp.int32),
        pltpu.VMEM((_B_EB,), jnp.int32),
        pltpu.VMEM((_B_EB,), jnp.int32),
        pltpu.VMEM((_B_EB,), jnp.float32),
        pltpu.VMEM((_B_NS, _SB), jnp.int32),
        pltpu.VMEM((_B_NS, _SB), jnp.int32),
        pltpu.VMEM((_SB, CW), jnp.float32),
        pltpu.VMEM((_SB, CW), jnp.float32),
        pltpu.SemaphoreType.DMA,
        pltpu.SemaphoreType.DMA,
        pltpu.SemaphoreType.DMA,
        pltpu.SemaphoreType.DMA,
    ],
)(_sc_layer2_body)


def kernel(x, edge_index, edge_type, W1, root1, bias1, W2, root2, bias2):
    src = edge_index[0].astype(jnp.int32)
    dst = edge_index[1].astype(jnp.int32)
    rel = edge_type.astype(jnp.int32)

    parts = _sc_counts(dst, rel)
    winv2d = _tc_merge(parts.reshape(NC * NS, NR // 128, 128))
    winv_flat = winv2d.reshape(NR)
    sums, wedge = _sc_sums(src, dst, rel, x, winv_flat)

    winv16 = jnp.repeat(winv2d.reshape(N, R), DIN, axis=1)
    h1 = _tc_layer1(sums.reshape(N, R * DIN), winv16, x,
                    W1.reshape(R * DIN, H), root1, bias1.reshape(1, H))

    w2x = jnp.concatenate(
        [W2.reshape(R, H, HC, CW), root2.reshape(1, H, HC, CW)],
        axis=0).transpose(2, 0, 1, 3).astype(jnp.bfloat16)
    hx = _tc_hall(h1, w2x, bias2.reshape(HC, 1, CW))

    return _sc_layer2(src, dst, rel, wedge, hx)
